# baseline (speedup 1.0000x reference)
import functools

import jax
import jax.numpy as jnp
from jax.experimental import pallas as pl
from jax.experimental.pallas import tpu as pltpu

B, R, N, D = 64, 29, 5000, 512
TOP_K = 5
NEG_INF = float("-inf")


def _region_kernel(q_ref, db_ref, vals_ref, idx_ref, sbuf_ref):
    r = pl.program_id(0)
    cur = jax.lax.rem(r, 2)
    prev = 1 - cur

    q = q_ref[0]
    db = db_ref[0]

    qn = jnp.sqrt(jnp.sum(q * q, axis=1, keepdims=True))
    qh = q / jnp.maximum(qn, 1e-12)

    dsq = db * db
    p = (dsq[:, 0:128] + dsq[:, 128:256]) + (dsq[:, 256:384] + dsq[:, 384:512])
    ones = jnp.ones((128, 8), dtype=jnp.float32)
    ssq = jax.lax.dot_general(
        p, ones, (((1,), (0,)), ((), ())),
        preferred_element_type=jnp.float32,
        precision=jax.lax.Precision.HIGHEST,
    )
    dbh = db / jnp.maximum(jnp.sqrt(ssq[:, 0:1]), 1e-12)

    sims = jax.lax.dot_general(
        qh.astype(jnp.bfloat16), dbh.astype(jnp.bfloat16),
        (((1,), (1,)), ((), ())),
        preferred_element_type=jnp.float32,
    )
    sbuf_ref[pl.ds(cur, 1)] = sims[None]

    s = sbuf_ref[pl.ds(prev, 1)][0]
    lane = jax.lax.broadcasted_iota(jnp.int32, (B, N), 1)
    vals = []
    idxs = []
    for _ in range(TOP_K):
        m = jnp.max(s, axis=1, keepdims=True)
        ix = jnp.min(jnp.where(s == m, lane, N), axis=1, keepdims=True)
        vals.append(m)
        idxs.append(ix)
        s = jnp.where(lane == ix, NEG_INF, s)
    vals_ref[0] = jnp.concatenate(vals, axis=1)
    idx_ref[0] = jnp.concatenate(idxs, axis=1)


@functools.partial(jax.jit, static_argnames=())
def _run(qT, db):
    grid = (R + 1,)
    vals_rbk, idx_rbk = pl.pallas_call(
        _region_kernel,
        grid=grid,
        in_specs=[
            pl.BlockSpec((1, B, D), lambda r: (jnp.minimum(r, R - 1), 0, 0)),
            pl.BlockSpec((1, N, D), lambda r: (jnp.minimum(r, R - 1), 0, 0)),
        ],
        out_specs=[
            pl.BlockSpec((1, B, TOP_K), lambda r: (jnp.maximum(r - 1, 0), 0, 0)),
            pl.BlockSpec((1, B, TOP_K), lambda r: (jnp.maximum(r - 1, 0), 0, 0)),
        ],
        out_shape=[
            jax.ShapeDtypeStruct((R, B, TOP_K), jnp.float32),
            jax.ShapeDtypeStruct((R, B, TOP_K), jnp.int32),
        ],
        scratch_shapes=[pltpu.VMEM((2, B, N), jnp.float32)],
        compiler_params=pltpu.CompilerParams(
            dimension_semantics=("arbitrary",),
        ),
    )(qT, db)
    return vals_rbk, idx_rbk


def kernel(query_visual_features, region_features_db, top_k):
    qT = jnp.transpose(query_visual_features, (1, 0, 2))
    vals_rbk, idx_rbk = _run(qT, region_features_db)
    top_vals = jnp.transpose(vals_rbk, (1, 0, 2))
    top_idx = jnp.transpose(idx_rbk, (1, 0, 2))
    similarity_scores = top_vals[..., 0]
    return top_vals, top_idx, similarity_scores


# speedup vs baseline: 1.0200x; 1.0200x over previous
import functools

import jax
import jax.numpy as jnp
from jax.experimental import pallas as pl
from jax.experimental.pallas import tpu as pltpu

B, R, N, D = 64, 29, 5000, 512
TOP_K = 5
NEG_INF = float("-inf")


def _region_kernel(q_ref, db_ref, vals_ref, idx_ref):
    q = q_ref[0]
    db = db_ref[0]

    qn = jnp.sqrt(jnp.sum(q * q, axis=1, keepdims=True))
    qh = q / jnp.maximum(qn, 1e-12)

    dsq = db * db
    p = (dsq[:, 0:128] + dsq[:, 128:256]) + (dsq[:, 256:384] + dsq[:, 384:512])
    ones = jnp.ones((128, 8), dtype=jnp.float32)
    ssq = jax.lax.dot_general(
        p, ones, (((1,), (0,)), ((), ())),
        preferred_element_type=jnp.float32,
        precision=jax.lax.Precision.HIGHEST,
    )
    dbh = db / jnp.maximum(jnp.sqrt(ssq[:, 0:1]), 1e-12)

    sims = jax.lax.dot_general(
        qh.astype(jnp.bfloat16), dbh.astype(jnp.bfloat16),
        (((1,), (1,)), ((), ())),
        preferred_element_type=jnp.float32,
    )

    lane = jax.lax.broadcasted_iota(jnp.int32, (B, N), 1)
    vals = []
    idxs = []
    s = sims
    for k in range(TOP_K):
        m = jnp.max(s, axis=1, keepdims=True)
        hit = s == m
        ix = jnp.min(jnp.where(hit, lane, N), axis=1, keepdims=True)
        vals.append(m)
        idxs.append(ix)
        if k + 1 < TOP_K:
            s = jnp.where(lane == ix, NEG_INF, s)
    vals_ref[0] = jnp.concatenate(vals, axis=1)
    idx_ref[0] = jnp.concatenate(idxs, axis=1)


@functools.partial(jax.jit, static_argnames=())
def _run(qT, db):
    grid = (R,)
    vals_rbk, idx_rbk = pl.pallas_call(
        _region_kernel,
        grid=grid,
        in_specs=[
            pl.BlockSpec((1, B, D), lambda r: (r, 0, 0)),
            pl.BlockSpec((1, N, D), lambda r: (r, 0, 0)),
        ],
        out_specs=[
            pl.BlockSpec((1, B, TOP_K), lambda r: (r, 0, 0)),
            pl.BlockSpec((1, B, TOP_K), lambda r: (r, 0, 0)),
        ],
        out_shape=[
            jax.ShapeDtypeStruct((R, B, TOP_K), jnp.float32),
            jax.ShapeDtypeStruct((R, B, TOP_K), jnp.int32),
        ],
        compiler_params=pltpu.CompilerParams(
            dimension_semantics=("parallel",),
        ),
    )(qT, db)
    return vals_rbk, idx_rbk


def kernel(query_visual_features, region_features_db, top_k):
    qT = jnp.transpose(query_visual_features, (1, 0, 2))
    vals_rbk, idx_rbk = _run(qT, region_features_db)
    top_vals = jnp.transpose(vals_rbk, (1, 0, 2))
    top_idx = jnp.transpose(idx_rbk, (1, 0, 2))
    similarity_scores = top_vals[..., 0]
    return top_vals, top_idx, similarity_scores


# speedup vs baseline: 1.2725x; 1.2475x over previous
import functools

import jax
import jax.numpy as jnp
from jax.experimental import pallas as pl
from jax.experimental.pallas import tpu as pltpu

B, R, N, D = 64, 29, 5000, 512
TOP_K = 5
NEG_INF = float("-inf")


def _region_kernel(q_ref, db_ref, vals_ref, idx_ref):
    q = q_ref[0]
    db = db_ref[0]

    qn = jnp.sqrt(jnp.sum(q * q, axis=1, keepdims=True))
    qh = q / jnp.maximum(qn, 1e-12)

    dsq = db * db
    p = (dsq[:, 0:128] + dsq[:, 128:256]) + (dsq[:, 256:384] + dsq[:, 384:512])
    hi = p.astype(jnp.bfloat16)
    r1 = p - hi.astype(jnp.float32)
    mid = r1.astype(jnp.bfloat16)
    r2 = r1 - mid.astype(jnp.float32)
    lo = r2.astype(jnp.bfloat16)
    ones = jnp.ones((128, 8), dtype=jnp.bfloat16)
    dn = (((1,), (0,)), ((), ()))
    ssq = (
        jax.lax.dot_general(hi, ones, dn, preferred_element_type=jnp.float32)
        + jax.lax.dot_general(mid, ones, dn, preferred_element_type=jnp.float32)
        + jax.lax.dot_general(lo, ones, dn, preferred_element_type=jnp.float32)
    )
    dbh = db / jnp.maximum(jnp.sqrt(ssq[:, 0:1]), 1e-12)

    sims = jax.lax.dot_general(
        qh.astype(jnp.bfloat16), dbh.astype(jnp.bfloat16),
        (((1,), (1,)), ((), ())),
        preferred_element_type=jnp.float32,
    )

    lane = jax.lax.broadcasted_iota(jnp.int32, (B, N), 1)
    vals = []
    idxs = []
    s = sims
    for k in range(TOP_K):
        m = jnp.max(s, axis=1, keepdims=True)
        hit = s == m
        ix = jnp.min(jnp.where(hit, lane, N), axis=1, keepdims=True)
        vals.append(m)
        idxs.append(ix)
        if k + 1 < TOP_K:
            s = jnp.where(lane == ix, NEG_INF, s)
    vals_ref[0] = jnp.concatenate(vals, axis=1)
    idx_ref[0] = jnp.concatenate(idxs, axis=1)


@functools.partial(jax.jit, static_argnames=())
def _run(qT, db):
    grid = (R,)
    vals_rbk, idx_rbk = pl.pallas_call(
        _region_kernel,
        grid=grid,
        in_specs=[
            pl.BlockSpec((1, B, D), lambda r: (r, 0, 0)),
            pl.BlockSpec((1, N, D), lambda r: (r, 0, 0)),
        ],
        out_specs=[
            pl.BlockSpec((1, B, TOP_K), lambda r: (r, 0, 0)),
            pl.BlockSpec((1, B, TOP_K), lambda r: (r, 0, 0)),
        ],
        out_shape=[
            jax.ShapeDtypeStruct((R, B, TOP_K), jnp.float32),
            jax.ShapeDtypeStruct((R, B, TOP_K), jnp.int32),
        ],
        compiler_params=pltpu.CompilerParams(
            dimension_semantics=("parallel",),
        ),
    )(qT, db)
    return vals_rbk, idx_rbk


def kernel(query_visual_features, region_features_db, top_k):
    qT = jnp.transpose(query_visual_features, (1, 0, 2))
    vals_rbk, idx_rbk = _run(qT, region_features_db)
    top_vals = jnp.transpose(vals_rbk, (1, 0, 2))
    top_idx = jnp.transpose(idx_rbk, (1, 0, 2))
    similarity_scores = top_vals[..., 0]
    return top_vals, top_idx, similarity_scores


# speedup vs baseline: 1.3470x; 1.0585x over previous
import functools

import jax
import jax.numpy as jnp
from jax.experimental import pallas as pl
from jax.experimental.pallas import tpu as pltpu

B, R, N, D = 64, 29, 5000, 512
TOP_K = 5
NEG_INF = float("-inf")


def _region_kernel(q_ref, db_ref, vals_ref, idx_ref):
    q = q_ref[0]
    db = db_ref[0]

    qn = jnp.sqrt(jnp.sum(q * q, axis=1, keepdims=True))
    qh = q / jnp.maximum(qn, 1e-12)

    dsq = db * db
    p = (dsq[:, 0:128] + dsq[:, 128:256]) + (dsq[:, 256:384] + dsq[:, 384:512])
    hi = p.astype(jnp.bfloat16)
    r1 = p - hi.astype(jnp.float32)
    mid = r1.astype(jnp.bfloat16)
    r2 = r1 - mid.astype(jnp.float32)
    lo = r2.astype(jnp.bfloat16)
    ones = jnp.ones((128, 8), dtype=jnp.bfloat16)
    dn = (((1,), (0,)), ((), ()))
    ssq = (
        jax.lax.dot_general(hi, ones, dn, preferred_element_type=jnp.float32)
        + jax.lax.dot_general(mid, ones, dn, preferred_element_type=jnp.float32)
        + jax.lax.dot_general(lo, ones, dn, preferred_element_type=jnp.float32)
    )
    dbh = db / jnp.maximum(jnp.sqrt(ssq[:, 0:1]), 1e-12)

    sims = jax.lax.dot_general(
        qh.astype(jnp.bfloat16), dbh.astype(jnp.bfloat16),
        (((1,), (1,)), ((), ())),
        preferred_element_type=jnp.float32,
    )

    lane = jax.lax.broadcasted_iota(jnp.int32, (B, N), 1)
    vals = []
    idxs = []
    s = sims
    for k in range(TOP_K):
        m = jnp.max(s, axis=1, keepdims=True)
        ix = jnp.argmax(s, axis=1, keepdims=True).astype(jnp.int32)
        vals.append(m)
        idxs.append(ix)
        if k + 1 < TOP_K:
            s = jnp.where(lane == ix, NEG_INF, s)
    vals_ref[0] = jnp.concatenate(vals, axis=1)
    idx_ref[0] = jnp.concatenate(idxs, axis=1)


@functools.partial(jax.jit, static_argnames=())
def _run(qT, db):
    grid = (R,)
    vals_rbk, idx_rbk = pl.pallas_call(
        _region_kernel,
        grid=grid,
        in_specs=[
            pl.BlockSpec((1, B, D), lambda r: (r, 0, 0)),
            pl.BlockSpec((1, N, D), lambda r: (r, 0, 0)),
        ],
        out_specs=[
            pl.BlockSpec((1, B, TOP_K), lambda r: (r, 0, 0)),
            pl.BlockSpec((1, B, TOP_K), lambda r: (r, 0, 0)),
        ],
        out_shape=[
            jax.ShapeDtypeStruct((R, B, TOP_K), jnp.float32),
            jax.ShapeDtypeStruct((R, B, TOP_K), jnp.int32),
        ],
        compiler_params=pltpu.CompilerParams(
            dimension_semantics=("parallel",),
        ),
    )(qT, db)
    return vals_rbk, idx_rbk


def kernel(query_visual_features, region_features_db, top_k):
    qT = jnp.transpose(query_visual_features, (1, 0, 2))
    vals_rbk, idx_rbk = _run(qT, region_features_db)
    top_vals = jnp.transpose(vals_rbk, (1, 0, 2))
    top_idx = jnp.transpose(idx_rbk, (1, 0, 2))
    similarity_scores = top_vals[..., 0]
    return top_vals, top_idx, similarity_scores
